# Initial kernel scaffold; baseline (speedup 1.0000x reference)
import functools

import jax
import jax.numpy as jnp
from jax import lax
from jax.experimental import pallas as pl
from jax.experimental.pallas import tpu as pltpu
from jax.experimental.pallas import tpu_sc as plsc

NC, NS, L = 2, 16, 16
NW = NC * NS
NEG = 0.2
BM = 2048


def _prologue_body(x_ref, w_ref, a2_ref, h_ref, ap_ref):
    h = jnp.dot(x_ref[...], w_ref[...], preferred_element_type=jnp.float32)
    h_ref[...] = h
    ap_ref[...] = jnp.dot(h, a2_ref[...], preferred_element_type=jnp.float32)


def _tc_prologue(xp, W, A2):
    NP, Fin = xp.shape
    H = W.shape[1]
    return pl.pallas_call(
        _prologue_body,
        grid=(NP // BM,),
        in_specs=[pl.BlockSpec((BM, Fin), lambda i: (i, 0)),
                  pl.BlockSpec((Fin, H), lambda i: (0, 0)),
                  pl.BlockSpec((H, 2), lambda i: (0, 0))],
        out_specs=[pl.BlockSpec((BM, H), lambda i: (i, 0)),
                   pl.BlockSpec((BM, 2), lambda i: (i, 0))],
        out_shape=[jax.ShapeDtypeStruct((NP, H), jnp.float32),
                   jax.ShapeDtypeStruct((NP, 2), jnp.float32)],
    )(xp, W, A2)


def _make_mid_body(n_valid):
    def _mid_body(p_ref, b_ref, w_ref, a2_ref, h_ref, ap_ref):
        z = p_ref[0] + p_ref[1] + b_ref[...]
        z = jnp.maximum(z, 0.0)
        rows = pl.program_id(0) * BM + lax.broadcasted_iota(
            jnp.int32, (BM, 1), 0)
        z = jnp.where(rows < n_valid, z, 0.0)
        h = jnp.dot(z, w_ref[...], preferred_element_type=jnp.float32)
        h_ref[...] = h
        ap_ref[...] = jnp.dot(h, a2_ref[...],
                              preferred_element_type=jnp.float32)
    return _mid_body


def _tc_mid(parts, b, W, A2, n_valid):
    NP, H = parts.shape[1], parts.shape[2]
    C = W.shape[1]
    return pl.pallas_call(
        _make_mid_body(n_valid),
        grid=(NP // BM,),
        in_specs=[pl.BlockSpec((2, BM, H), lambda i: (0, i, 0)),
                  pl.BlockSpec((1, H), lambda i: (0, 0)),
                  pl.BlockSpec((H, C), lambda i: (0, 0)),
                  pl.BlockSpec((C, 2), lambda i: (0, 0))],
        out_specs=[pl.BlockSpec((BM, C), lambda i: (i, 0)),
                   pl.BlockSpec((BM, 2), lambda i: (i, 0))],
        out_shape=[jax.ShapeDtypeStruct((NP, C), jnp.float32),
                   jax.ShapeDtypeStruct((NP, 2), jnp.float32)],
    )(parts, b, W, A2)


def _final_body(p_ref, b_ref, o_ref):
    o_ref[...] = p_ref[0] + p_ref[1] + b_ref[...]


def _tc_final(parts, b):
    NP, C = parts.shape[1], parts.shape[2]
    return pl.pallas_call(
        _final_body,
        grid=(NP // BM,),
        in_specs=[pl.BlockSpec((2, BM, C), lambda i: (0, i, 0)),
                  pl.BlockSpec((1, C), lambda i: (0, 0))],
        out_specs=pl.BlockSpec((BM, C), lambda i: (i, 0)),
        out_shape=jax.ShapeDtypeStruct((NP, C), jnp.float32),
    )(parts, b)


def _make_sc_layer(NP, F, CH):
    nS = NP // NS
    mesh = plsc.VectorSubcoreMesh(core_axis_name="c", subcore_axis_name="s")

    @functools.partial(
        pl.kernel,
        out_type=jax.ShapeDtypeStruct((NC, NP, F), jnp.float32),
        mesh=mesh,
        scratch_types=[
            pltpu.VMEM_SHARED((NP,), jnp.float32),
            pltpu.VMEM_SHARED((NP,), jnp.float32),
            pltpu.VMEM_SHARED((NP, F), jnp.float32),
            pltpu.VMEM((NP,), jnp.float32),
            pltpu.VMEM((NP,), jnp.float32),
            pltpu.VMEM((NP,), jnp.float32),
            pltpu.VMEM((CH, 128), jnp.int32),
            pltpu.VMEM((CH, 128), jnp.int32),
            pltpu.VMEM((128,), jnp.float32),
            pltpu.VMEM((128, F), jnp.float32),
            pltpu.VMEM((nS,), jnp.float32),
            pltpu.SemaphoreType.DMA,
        ],
    )
    def sc_layer(h_hbm, asrc_hbm, adst_hbm, srcE, dstE, parts,
                 den_sh, rden_sh, out_sh, asrc_v, adst_v, rden_v,
                 srcC_v, dstC_v, wbuf, rowbuf, slice_v, gsem):
        cid = lax.axis_index("c")
        sid = lax.axis_index("s")
        wid = cid * NS + sid
        base = sid * nS

        pltpu.sync_copy(asrc_hbm, asrc_v)
        pltpu.sync_copy(adst_hbm, adst_v)

        def zrow(i, _):
            for q in range(F // L):
                rowbuf[i, pl.ds(q * L, L)] = jnp.zeros((L,), jnp.float32)
            return 0
        lax.fori_loop(0, 128, zrow, 0)

        def zsl(i, _):
            slice_v[pl.ds(i * L, L)] = jnp.zeros((L,), jnp.float32)
            return 0
        lax.fori_loop(0, nS // L, zsl, 0)

        pltpu.sync_copy(slice_v, den_sh.at[pl.ds(base, nS)])
        for k in range(nS // 128):
            pltpu.sync_copy(rowbuf, out_sh.at[pl.ds(base + k * 128, 128)])
        plsc.subcore_barrier()

        def mx(i, m):
            return jnp.maximum(m, asrc_v[pl.ds(i * L, L)])
        m16 = lax.fori_loop(0, NP // L, mx,
                            jnp.full((L,), -3.0e38, jnp.float32))
        amax = jnp.max(m16)

        def edge_weights(r, use_rden):
            for q in range(128 // L):
                sv = srcC_v[r, pl.ds(q * L, L)]
                dv = dstC_v[r, pl.ds(q * L, L)]
                a_s = plsc.load_gather(asrc_v, [sv])
                a_d = plsc.load_gather(adst_v, [dv])
                al = a_s + a_d
                al = jnp.maximum(al, NEG * al)
                sh = a_d + amax
                sh = jnp.maximum(sh, NEG * sh)
                e = jnp.exp(al - sh)
                if use_rden:
                    e = e * plsc.load_gather(rden_v, [dv])
                wbuf[pl.ds(q * L, L)] = e

        def passA_block(blk):
            pltpu.sync_copy(srcE.at[blk], srcC_v)
            pltpu.sync_copy(dstE.at[blk], dstC_v)

            def rowA(r, _):
                edge_weights(r, False)
                pltpu.sync_copy(wbuf, den_sh.at[dstC_v.at[r]], add=True)
                return 0
            lax.fori_loop(0, CH, rowA, 0)

        passA_block(sid * 2)
        passA_block(sid * 2 + 1)
        plsc.subcore_barrier()

        pltpu.sync_copy(den_sh.at[pl.ds(base, nS)], slice_v)

        def rcp(i, _):
            v = slice_v[pl.ds(i * L, L)]
            slice_v[pl.ds(i * L, L)] = 1.0 / (v + 1e-30)
            return 0
        lax.fori_loop(0, nS // L, rcp, 0)
        pltpu.sync_copy(slice_v, rden_sh.at[pl.ds(base, nS)])
        plsc.subcore_barrier()
        pltpu.sync_copy(rden_sh, rden_v)

        pltpu.sync_copy(srcE.at[wid], srcC_v)
        pltpu.sync_copy(dstE.at[wid], dstC_v)

        def chunk(g, _):
            cp = pltpu.async_copy(h_hbm.at[srcC_v.at[g]], rowbuf, gsem)
            edge_weights(g, True)
            cp.wait()

            def rgrp(t, _):
                for rr in range(8):
                    row = t * 8 + rr
                    ws = wbuf[row]
                    for q in range(F // L):
                        rowbuf[row, pl.ds(q * L, L)] = (
                            rowbuf[row, pl.ds(q * L, L)] * ws)
                return 0
            lax.fori_loop(0, 16, rgrp, 0)
            pltpu.sync_copy(rowbuf, out_sh.at[dstC_v.at[g]], add=True)
            return 0
        lax.fori_loop(0, CH, chunk, 0)
        plsc.subcore_barrier()

        pltpu.sync_copy(out_sh.at[pl.ds(base, nS)],
                        parts.at[cid, pl.ds(base, nS)])

    return sc_layer


def kernel(x, edge_index, W1, att_src1, att_dst1, b1,
           W2, att_src2, att_dst2, b2):
    N, Fin = x.shape
    E = edge_index.shape[1]
    H = W1.shape[1]
    C = W2.shape[1]

    NP = (N // (NS * 128) + 1) * (NS * 128)
    Et = E + N
    EP = -(-Et // (NW * 128)) * (NW * 128)
    CH = EP // (NW * 128)
    npad = EP - Et

    ei = edge_index.astype(jnp.int32)
    loops = jnp.arange(N, dtype=jnp.int32)
    pad_src = jnp.arange(npad, dtype=jnp.int32) % N
    pad_dst = N + jnp.arange(npad, dtype=jnp.int32) % (NP - N)
    src = jnp.concatenate([ei[0], loops, pad_src]).reshape(NW, CH, 128)
    dst = jnp.concatenate([ei[1], loops, pad_dst]).reshape(NW, CH, 128)

    xp = jnp.zeros((NP, Fin), jnp.float32).at[:N].set(x)
    A21 = jnp.stack([att_src1, att_dst1], axis=1)
    A22 = jnp.stack([att_src2, att_dst2], axis=1)

    h1, ap1 = _tc_prologue(xp, W1, A21)
    sc1 = _make_sc_layer(NP, H, CH)
    parts1 = sc1(h1, ap1[:, 0], ap1[:, 1], src, dst)

    h2, ap2 = _tc_mid(parts1, b1.reshape(1, H), W2, A22, N)
    sc2 = _make_sc_layer(NP, C, CH)
    parts2 = sc2(h2, ap2[:, 0], ap2[:, 1], src, dst)

    out = _tc_final(parts2, b2.reshape(1, C))
    return out[:N], edge_index


# speedup vs baseline: 22.8270x; 22.8270x over previous
import functools

import jax
import jax.numpy as jnp
from jax import lax
from jax.experimental import pallas as pl
from jax.experimental.pallas import tpu as pltpu
from jax.experimental.pallas import tpu_sc as plsc

NC, NS, L = 2, 16, 16
NW = NC * NS
NEG = 0.2
BM = 2048


def _prologue_body(x_ref, w_ref, a2_ref, h_ref, ap_ref):
    h = jnp.dot(x_ref[...], w_ref[...], preferred_element_type=jnp.float32)
    h_ref[...] = h
    ap_ref[...] = jnp.dot(h, a2_ref[...], preferred_element_type=jnp.float32)


def _tc_prologue(xp, W, A2):
    NP, Fin = xp.shape
    H = W.shape[1]
    return pl.pallas_call(
        _prologue_body,
        grid=(NP // BM,),
        in_specs=[pl.BlockSpec((BM, Fin), lambda i: (i, 0)),
                  pl.BlockSpec((Fin, H), lambda i: (0, 0)),
                  pl.BlockSpec((H, 2), lambda i: (0, 0))],
        out_specs=[pl.BlockSpec((BM, H), lambda i: (i, 0)),
                   pl.BlockSpec((BM, 2), lambda i: (i, 0))],
        out_shape=[jax.ShapeDtypeStruct((NP, H), jnp.float32),
                   jax.ShapeDtypeStruct((NP, 2), jnp.float32)],
    )(xp, W, A2)


def _make_mid_body(n_valid):
    def _mid_body(p_ref, b_ref, w_ref, a2_ref, h_ref, ap_ref):
        z = p_ref[0] + p_ref[1] + b_ref[...]
        z = jnp.maximum(z, 0.0)
        rows = pl.program_id(0) * BM + lax.broadcasted_iota(
            jnp.int32, (BM, 1), 0)
        z = jnp.where(rows < n_valid, z, 0.0)
        h = jnp.dot(z, w_ref[...], preferred_element_type=jnp.float32)
        h_ref[...] = h
        ap_ref[...] = jnp.dot(h, a2_ref[...],
                              preferred_element_type=jnp.float32)
    return _mid_body


def _tc_mid(parts, b, W, A2, n_valid):
    NP, H = parts.shape[1], parts.shape[2]
    C = W.shape[1]
    return pl.pallas_call(
        _make_mid_body(n_valid),
        grid=(NP // BM,),
        in_specs=[pl.BlockSpec((2, BM, H), lambda i: (0, i, 0)),
                  pl.BlockSpec((1, H), lambda i: (0, 0)),
                  pl.BlockSpec((H, C), lambda i: (0, 0)),
                  pl.BlockSpec((C, 2), lambda i: (0, 0))],
        out_specs=[pl.BlockSpec((BM, C), lambda i: (i, 0)),
                   pl.BlockSpec((BM, 2), lambda i: (i, 0))],
        out_shape=[jax.ShapeDtypeStruct((NP, C), jnp.float32),
                   jax.ShapeDtypeStruct((NP, 2), jnp.float32)],
    )(parts, b, W, A2)


def _final_body(p_ref, b_ref, o_ref):
    o_ref[...] = p_ref[0] + p_ref[1] + b_ref[...]


def _tc_final(parts, b):
    NP, C = parts.shape[1], parts.shape[2]
    return pl.pallas_call(
        _final_body,
        grid=(NP // BM,),
        in_specs=[pl.BlockSpec((2, BM, C), lambda i: (0, i, 0)),
                  pl.BlockSpec((1, C), lambda i: (0, 0))],
        out_specs=pl.BlockSpec((BM, C), lambda i: (i, 0)),
        out_shape=jax.ShapeDtypeStruct((NP, C), jnp.float32),
    )(parts, b)


def _make_sc_layer(NP, F, CH):
    nS = NP // NS
    mesh = plsc.VectorSubcoreMesh(core_axis_name="c", subcore_axis_name="s")

    @functools.partial(
        pl.kernel,
        out_type=jax.ShapeDtypeStruct((NC, NP, F), jnp.float32),
        mesh=mesh,
        compiler_params=pltpu.CompilerParams(
            needs_layout_passes=False,
            use_tc_tiling_on_sc=(F >= 128)),
        scratch_types=[
            pltpu.VMEM_SHARED((NP,), jnp.float32),
            pltpu.VMEM_SHARED((NP, F), jnp.float32),
            pltpu.VMEM((NP,), jnp.float32),
            pltpu.VMEM((NP,), jnp.float32),
            pltpu.VMEM((2, 128), jnp.int32),
            pltpu.VMEM((2, 128), jnp.int32),
            pltpu.VMEM((128,), jnp.float32),
            pltpu.VMEM((128, F), jnp.float32),
            pltpu.VMEM((nS,), jnp.float32),
            pltpu.SemaphoreType.DMA,
        ],
    )
    def sc_layer(h_hbm, asrc_hbm, adst_hbm, srcE, dstE, parts,
                 den_sh, out_sh, asrc_v, adst_v,
                 sring, dring, wbuf, rowbuf, slice_v, gsem):
        cid = lax.axis_index("c")
        sid = lax.axis_index("s")
        wid = cid * NS + sid
        base = sid * nS

        pltpu.sync_copy(asrc_hbm, asrc_v)
        pltpu.sync_copy(adst_hbm, adst_v)

        def zrow(i, _):
            for q in range(F // L):
                rowbuf[i, pl.ds(q * L, L)] = jnp.zeros((L,), jnp.float32)
            return 0
        lax.fori_loop(0, 128, zrow, 0)

        def zsl(i, _):
            slice_v[pl.ds(i * L, L)] = jnp.zeros((L,), jnp.float32)
            return 0
        lax.fori_loop(0, nS // L, zsl, 0)

        pltpu.sync_copy(slice_v, den_sh.at[pl.ds(base, nS)])
        for k in range(nS // 128):
            pltpu.sync_copy(rowbuf, out_sh.at[pl.ds(base + k * 128, 128)])
        plsc.subcore_barrier()

        def mx(i, m):
            return jnp.maximum(m, asrc_v[pl.ds(i * L, L)])
        m16 = lax.fori_loop(0, NP // L, mx,
                            jnp.full((L,), -3.0e38, jnp.float32))
        amax = m16[0]
        for i in range(1, L):
            amax = jnp.maximum(amax, m16[i])

        def edge_weights():
            for q in range(128 // L):
                sv = sring[0, pl.ds(q * L, L)]
                dv = dring[0, pl.ds(q * L, L)]
                a_s = plsc.load_gather(asrc_v, [sv])
                a_d = plsc.load_gather(adst_v, [dv])
                al = a_s + a_d
                al = jnp.maximum(al, NEG * al)
                sh = a_d + amax
                sh = jnp.maximum(sh, NEG * sh)
                wbuf[pl.ds(q * L, L)] = jnp.exp(al - sh)

        def passA_block(blk):
            def rowA(g, _):
                pltpu.sync_copy(srcE.at[blk, g], sring.at[0])
                pltpu.sync_copy(dstE.at[blk, g], dring.at[0])
                edge_weights()
                pltpu.sync_copy(wbuf, den_sh.at[dring.at[0]], add=True)
                return 0
            lax.fori_loop(0, CH, rowA, 0)

        passA_block(sid * 2)
        passA_block(sid * 2 + 1)
        plsc.subcore_barrier()

        def chunk(g, _):
            pltpu.sync_copy(srcE.at[wid, g], sring.at[0])
            pltpu.sync_copy(dstE.at[wid, g], dring.at[0])
            cp = pltpu.async_copy(h_hbm.at[sring.at[0]], rowbuf, gsem)
            edge_weights()
            cp.wait()

            def rgrp(t, _):
                wv = wbuf[pl.ds(t * L, L)]
                for rr in range(L):
                    row = t * L + rr
                    ws = wv[rr]
                    for q in range(F // L):
                        rowbuf[row, pl.ds(q * L, L)] = (
                            rowbuf[row, pl.ds(q * L, L)] * ws)
                return 0
            lax.fori_loop(0, 128 // L, rgrp, 0)
            pltpu.sync_copy(rowbuf, out_sh.at[dring.at[0]], add=True)
            return 0
        lax.fori_loop(0, CH, chunk, 0)
        plsc.subcore_barrier()

        pltpu.sync_copy(den_sh.at[pl.ds(base, nS)], slice_v)

        def rcp(i, _):
            v = slice_v[pl.ds(i * L, L)]
            slice_v[pl.ds(i * L, L)] = 1.0 / (v + 1e-30)
            return 0
        lax.fori_loop(0, nS // L, rcp, 0)

        def ep_batch(k, _):
            pltpu.sync_copy(out_sh.at[pl.ds(base + k * 128, 128)], rowbuf)

            def ep_grp(t, _):
                rv = slice_v[pl.ds(k * 128 + t * L, L)]
                for rr in range(L):
                    row = t * L + rr
                    ws = rv[rr]
                    for q in range(F // L):
                        rowbuf[row, pl.ds(q * L, L)] = (
                            rowbuf[row, pl.ds(q * L, L)] * ws)
                return 0
            lax.fori_loop(0, 128 // L, ep_grp, 0)
            pltpu.sync_copy(rowbuf,
                            parts.at[cid, pl.ds(base + k * 128, 128)])
            return 0
        lax.fori_loop(0, nS // 128, ep_batch, 0)

    return sc_layer


def kernel(x, edge_index, W1, att_src1, att_dst1, b1,
           W2, att_src2, att_dst2, b2):
    N, Fin = x.shape
    E = edge_index.shape[1]
    H = W1.shape[1]
    C = W2.shape[1]

    NP = (N // (NS * 128) + 1) * (NS * 128)
    Et = E + N
    EP = -(-Et // (NW * 128)) * (NW * 128)
    CH = EP // (NW * 128)
    npad = EP - Et

    ei = edge_index.astype(jnp.int32)
    loops = jnp.arange(N, dtype=jnp.int32)
    pad_src = jnp.arange(npad, dtype=jnp.int32) % N
    pad_dst = N + jnp.arange(npad, dtype=jnp.int32) % (NP - N)
    src = jnp.concatenate([ei[0], loops, pad_src]).reshape(NW, CH, 128)
    dst = jnp.concatenate([ei[1], loops, pad_dst]).reshape(NW, CH, 128)

    xp = jnp.zeros((NP, Fin), jnp.float32).at[:N].set(x)
    A21 = jnp.stack([att_src1, att_dst1], axis=1)
    A22 = jnp.stack([att_src2, att_dst2], axis=1)

    h1, ap1 = _tc_prologue(xp, W1, A21)
    sc1 = _make_sc_layer(NP, H, CH)
    parts1 = sc1(h1, ap1[:, 0], ap1[:, 1], src, dst)

    h2, ap2 = _tc_mid(parts1, b1.reshape(1, H), W2, A22, N)
    sc2 = _make_sc_layer(NP, C, CH)
    parts2 = sc2(h2, ap2[:, 0], ap2[:, 1], src, dst)

    out = _tc_final(parts2, b2.reshape(1, C))
    return out[:N], edge_index


# speedup vs baseline: 39.2461x; 1.7193x over previous
import functools

import jax
import jax.numpy as jnp
from jax import lax
from jax.experimental import pallas as pl
from jax.experimental.pallas import tpu as pltpu
from jax.experimental.pallas import tpu_sc as plsc

NC, NS, L = 2, 16, 16
NW = NC * NS
NEG = 0.2
BM = 2048


def _prologue_body(x_ref, w_ref, a2_ref, h_ref, ap_ref):
    h = jnp.dot(x_ref[...], w_ref[...], preferred_element_type=jnp.float32)
    hf = h.shape[1] // 2
    h_ref[0] = h[:, :hf]
    h_ref[1] = h[:, hf:]
    ap_ref[...] = jnp.dot(h, a2_ref[...], preferred_element_type=jnp.float32)


def _tc_prologue(xp, W, A2):
    NP, Fin = xp.shape
    H = W.shape[1]
    return pl.pallas_call(
        _prologue_body,
        grid=(NP // BM,),
        in_specs=[pl.BlockSpec((BM, Fin), lambda i: (i, 0)),
                  pl.BlockSpec((Fin, H), lambda i: (0, 0)),
                  pl.BlockSpec((H, 2), lambda i: (0, 0))],
        out_specs=[pl.BlockSpec((2, BM, H // 2), lambda i: (0, i, 0)),
                   pl.BlockSpec((BM, 2), lambda i: (i, 0))],
        out_shape=[jax.ShapeDtypeStruct((2, NP, H // 2), jnp.float32),
                   jax.ShapeDtypeStruct((NP, 2), jnp.float32)],
    )(xp, W, A2)


def _make_mid_body(n_valid):
    def _mid_body(p_ref, d_ref, b_ref, w_ref, a2_ref, h_ref, ap_ref):
        agg = jnp.concatenate([p_ref[0], p_ref[1]], axis=1)
        rden = (1.0 / (d_ref[0] + 1e-30)).reshape(-1, 1)
        z = jnp.maximum(agg * rden + b_ref[...], 0.0)
        rows = pl.program_id(0) * BM + lax.broadcasted_iota(
            jnp.int32, (BM, 1), 0)
        z = jnp.where(rows < n_valid, z, 0.0)
        h = jnp.dot(z, w_ref[...], preferred_element_type=jnp.float32)
        h_ref[...] = h
        ap_ref[...] = jnp.dot(h, a2_ref[...],
                              preferred_element_type=jnp.float32)
    return _mid_body


def _tc_mid(parts, dens, b, W, A2, n_valid):
    NP, Fh = parts.shape[1], parts.shape[2]
    H = 2 * Fh
    C = W.shape[1]
    return pl.pallas_call(
        _make_mid_body(n_valid),
        grid=(NP // BM,),
        in_specs=[pl.BlockSpec((2, BM, Fh), lambda i: (0, i, 0)),
                  pl.BlockSpec((2, BM), lambda i: (0, i)),
                  pl.BlockSpec((1, H), lambda i: (0, 0)),
                  pl.BlockSpec((H, C), lambda i: (0, 0)),
                  pl.BlockSpec((C, 2), lambda i: (0, 0))],
        out_specs=[pl.BlockSpec((BM, C), lambda i: (i, 0)),
                   pl.BlockSpec((BM, 2), lambda i: (i, 0))],
        out_shape=[jax.ShapeDtypeStruct((NP, C), jnp.float32),
                   jax.ShapeDtypeStruct((NP, 2), jnp.float32)],
    )(parts, dens, b, W, A2)


def _final_body(p_ref, d_ref, b_ref, o_ref):
    rden = (1.0 / (d_ref[0] + d_ref[1] + 1e-30)).reshape(-1, 1)
    o_ref[...] = (p_ref[0] + p_ref[1]) * rden + b_ref[...]


def _tc_final(parts, dens, b):
    NP, C = parts.shape[1], parts.shape[2]
    return pl.pallas_call(
        _final_body,
        grid=(NP // BM,),
        in_specs=[pl.BlockSpec((2, BM, C), lambda i: (0, i, 0)),
                  pl.BlockSpec((2, BM), lambda i: (0, i)),
                  pl.BlockSpec((1, C), lambda i: (0, 0))],
        out_specs=pl.BlockSpec((BM, C), lambda i: (i, 0)),
        out_shape=jax.ShapeDtypeStruct((NP, C), jnp.float32),
    )(parts, dens, b)


def _make_sc_layer(NP, Fh, CH, bpt, off):
    nS = NP // NS
    mesh = plsc.VectorSubcoreMesh(core_axis_name="c", subcore_axis_name="s")

    @functools.partial(
        pl.kernel,
        out_type=[jax.ShapeDtypeStruct((NC, NP, Fh), jnp.float32),
                  jax.ShapeDtypeStruct((NC, NP), jnp.float32)],
        mesh=mesh,
        compiler_params=pltpu.CompilerParams(
            needs_layout_passes=False,
            use_tc_tiling_on_sc=(Fh >= 128)),
        scratch_types=[
            pltpu.VMEM_SHARED((NP,), jnp.float32),
            pltpu.VMEM_SHARED((NP, Fh), jnp.float32),
            pltpu.VMEM((NP,), jnp.float32),
            pltpu.VMEM((NP,), jnp.float32),
            pltpu.VMEM((CH, 128), jnp.int32),
            pltpu.VMEM((CH, 128), jnp.int32),
            pltpu.VMEM((128,), jnp.float32),
            pltpu.VMEM((128, Fh), jnp.float32),
            pltpu.SemaphoreType.DMA,
        ],
    )
    def sc_layer(h2d, asrc_hbm, adst_hbm, srcI, dstI, parts, dens,
                 den_sh, out_sh, asrc_v, adst_v,
                 srcP, dstP, wbuf, rowbuf, gsem):
        cid = lax.axis_index("c")
        sid = lax.axis_index("s")
        wid = cid * NS + sid
        base = sid * nS
        coff = cid * off

        pltpu.sync_copy(asrc_hbm, asrc_v)
        pltpu.sync_copy(adst_hbm, adst_v)

        def zrow(i, _):
            for q in range(Fh // L):
                rowbuf[i, pl.ds(q * L, L)] = jnp.zeros((L,), jnp.float32)
            return 0
        lax.fori_loop(0, 128, zrow, 0)
        for q in range(128 // L):
            wbuf[pl.ds(q * L, L)] = jnp.zeros((L,), jnp.float32)
        for k in range(nS // 128):
            pltpu.sync_copy(rowbuf, out_sh.at[pl.ds(base + k * 128, 128)])
            pltpu.sync_copy(wbuf, den_sh.at[pl.ds(base + k * 128, 128)])
        plsc.subcore_barrier()

        def mx(i, m):
            return jnp.maximum(m, asrc_v[pl.ds(i * L, L)])
        m16 = lax.fori_loop(0, NP // L, mx,
                            jnp.full((L,), -3.0e38, jnp.float32))
        amax = m16[0]
        for i in range(1, L):
            amax = jnp.maximum(amax, m16[i])

        def sweep_block(blk):
            pltpu.sync_copy(srcI.at[cid, blk], srcP)
            pltpu.sync_copy(dstI.at[blk], dstP)

            def chunk(g, _):
                cp = pltpu.async_copy(h2d.at[srcP.at[g]], rowbuf, gsem)
                for q in range(128 // L):
                    sv = srcP[g, pl.ds(q * L, L)] - coff
                    dv = dstP[g, pl.ds(q * L, L)]
                    a_s = plsc.load_gather(asrc_v, [sv])
                    a_d = plsc.load_gather(adst_v, [dv])
                    al = a_s + a_d
                    al = jnp.maximum(al, NEG * al)
                    sh = a_d + amax
                    sh = jnp.maximum(sh, NEG * sh)
                    wbuf[pl.ds(q * L, L)] = jnp.exp(al - sh)
                pltpu.sync_copy(wbuf, den_sh.at[dstP.at[g]], add=True)
                cp.wait()

                def rgrp(t, _):
                    wv = wbuf[pl.ds(t * L, L)]
                    for rr in range(L):
                        row = t * L + rr
                        ws = wv[rr]
                        for q in range(Fh // L):
                            rowbuf[row, pl.ds(q * L, L)] = (
                                rowbuf[row, pl.ds(q * L, L)] * ws)
                    return 0
                lax.fori_loop(0, 128 // L, rgrp, 0)
                pltpu.sync_copy(rowbuf, out_sh.at[dstP.at[g]], add=True)
                return 0
            lax.fori_loop(0, CH, chunk, 0)

        if bpt == 1:
            sweep_block(wid)
        else:
            for bb in range(bpt):
                sweep_block(bpt * sid + bb)
        plsc.subcore_barrier()

        pltpu.sync_copy(out_sh.at[pl.ds(base, nS)],
                        parts.at[cid, pl.ds(base, nS)])
        pltpu.sync_copy(den_sh.at[pl.ds(base, nS)],
                        dens.at[cid, pl.ds(base, nS)])

    return sc_layer


def kernel(x, edge_index, W1, att_src1, att_dst1, b1,
           W2, att_src2, att_dst2, b2):
    N, Fin = x.shape
    E = edge_index.shape[1]
    H = W1.shape[1]
    C = W2.shape[1]

    NP = (N // (NS * 128) + 1) * (NS * 128)
    Et = E + N
    EP = -(-Et // (NW * 128)) * (NW * 128)
    CH = EP // (NW * 128)
    npad = EP - Et

    ei = edge_index.astype(jnp.int32)
    loops = jnp.arange(N, dtype=jnp.int32)
    pad_src = jnp.arange(npad, dtype=jnp.int32) % N
    pad_dst = N + jnp.arange(npad, dtype=jnp.int32) % (NP - N)
    src = jnp.concatenate([ei[0], loops, pad_src]).reshape(NW, CH, 128)
    dst = jnp.concatenate([ei[1], loops, pad_dst]).reshape(NW, CH, 128)
    src1 = jnp.stack([src, src + NP])
    src2 = jnp.stack([src, src])

    xp = jnp.zeros((NP, Fin), jnp.float32).at[:N].set(x)
    A21 = jnp.stack([att_src1, att_dst1], axis=1)
    A22 = jnp.stack([att_src2, att_dst2], axis=1)

    h1s, ap1 = _tc_prologue(xp, W1, A21)
    sc1 = _make_sc_layer(NP, H // 2, CH, bpt=2, off=NP)
    parts1, dens1 = sc1(h1s.reshape(2 * NP, H // 2),
                        ap1[:, 0], ap1[:, 1], src1, dst)

    h2, ap2 = _tc_mid(parts1, dens1, b1.reshape(1, H), W2, A22, N)
    sc2 = _make_sc_layer(NP, C, CH, bpt=1, off=0)
    parts2, dens2 = sc2(h2, ap2[:, 0], ap2[:, 1], src2, dst)

    out = _tc_final(parts2, dens2, b2.reshape(1, C))
    return out[:N], edge_index


# speedup vs baseline: 61.6817x; 1.5717x over previous
import functools

import jax
import jax.numpy as jnp
from jax import lax
from jax.experimental import pallas as pl
from jax.experimental.pallas import tpu as pltpu
from jax.experimental.pallas import tpu_sc as plsc

NC, NS, L = 2, 16, 16
NW = NC * NS
NEG = 0.2
BM = 2048


def _prologue_body(x_ref, w_ref, a2_ref, h_ref, ap_ref):
    h = jnp.dot(x_ref[...], w_ref[...], preferred_element_type=jnp.float32)
    hf = h.shape[1] // 2
    h_ref[0] = h[:, :hf]
    h_ref[1] = h[:, hf:]
    ap_ref[...] = jnp.dot(h, a2_ref[...], preferred_element_type=jnp.float32)


def _tc_prologue(xp, W, A2):
    NP, Fin = xp.shape
    H = W.shape[1]
    return pl.pallas_call(
        _prologue_body,
        grid=(NP // BM,),
        in_specs=[pl.BlockSpec((BM, Fin), lambda i: (i, 0)),
                  pl.BlockSpec((Fin, H), lambda i: (0, 0)),
                  pl.BlockSpec((H, 2), lambda i: (0, 0))],
        out_specs=[pl.BlockSpec((2, BM, H // 2), lambda i: (0, i, 0)),
                   pl.BlockSpec((BM, 2), lambda i: (i, 0))],
        out_shape=[jax.ShapeDtypeStruct((2, NP, H // 2), jnp.float32),
                   jax.ShapeDtypeStruct((NP, 2), jnp.float32)],
    )(xp, W, A2)


def _make_mid_body(n_valid):
    def _mid_body(p_ref, d_ref, b_ref, w_ref, a2_ref, h_ref, ap_ref):
        agg = jnp.concatenate([p_ref[0], p_ref[1]], axis=1)
        rden = (1.0 / (d_ref[0] + 1e-30)).reshape(-1, 1)
        z = jnp.maximum(agg * rden + b_ref[...], 0.0)
        rows = pl.program_id(0) * BM + lax.broadcasted_iota(
            jnp.int32, (BM, 1), 0)
        z = jnp.where(rows < n_valid, z, 0.0)
        h = jnp.dot(z, w_ref[...], preferred_element_type=jnp.float32)
        h_ref[...] = h
        ap_ref[...] = jnp.dot(h, a2_ref[...],
                              preferred_element_type=jnp.float32)
    return _mid_body


def _tc_mid(parts, dens, b, W, A2, n_valid):
    NP, Fh = parts.shape[1], parts.shape[2]
    H = 2 * Fh
    C = W.shape[1]
    return pl.pallas_call(
        _make_mid_body(n_valid),
        grid=(NP // BM,),
        in_specs=[pl.BlockSpec((2, BM, Fh), lambda i: (0, i, 0)),
                  pl.BlockSpec((2, BM), lambda i: (0, i)),
                  pl.BlockSpec((1, H), lambda i: (0, 0)),
                  pl.BlockSpec((H, C), lambda i: (0, 0)),
                  pl.BlockSpec((C, 2), lambda i: (0, 0))],
        out_specs=[pl.BlockSpec((BM, C), lambda i: (i, 0)),
                   pl.BlockSpec((BM, 2), lambda i: (i, 0))],
        out_shape=[jax.ShapeDtypeStruct((NP, C), jnp.float32),
                   jax.ShapeDtypeStruct((NP, 2), jnp.float32)],
    )(parts, dens, b, W, A2)


def _final_body(p_ref, d_ref, b_ref, o_ref):
    rden = (1.0 / (d_ref[0] + d_ref[1] + 1e-30)).reshape(-1, 1)
    o_ref[...] = (p_ref[0] + p_ref[1]) * rden + b_ref[...]


def _tc_final(parts, dens, b):
    NP, C = parts.shape[1], parts.shape[2]
    return pl.pallas_call(
        _final_body,
        grid=(NP // BM,),
        in_specs=[pl.BlockSpec((2, BM, C), lambda i: (0, i, 0)),
                  pl.BlockSpec((2, BM), lambda i: (0, i)),
                  pl.BlockSpec((1, C), lambda i: (0, 0))],
        out_specs=pl.BlockSpec((BM, C), lambda i: (i, 0)),
        out_shape=jax.ShapeDtypeStruct((NP, C), jnp.float32),
    )(parts, dens, b)


def _make_sc_layer(NP, Fh, CH, bpt, off):
    nS = NP // NS
    mesh = plsc.VectorSubcoreMesh(core_axis_name="c", subcore_axis_name="s")

    @functools.partial(
        pl.kernel,
        out_type=[jax.ShapeDtypeStruct((NC, NP, Fh), jnp.float32),
                  jax.ShapeDtypeStruct((NC, NP), jnp.float32)],
        mesh=mesh,
        compiler_params=pltpu.CompilerParams(
            needs_layout_passes=False,
            use_tc_tiling_on_sc=(Fh >= 128)),
        scratch_types=[
            pltpu.VMEM_SHARED((NP,), jnp.float32),
            pltpu.VMEM_SHARED((NP, Fh), jnp.float32),
            pltpu.VMEM((NP,), jnp.float32),
            pltpu.VMEM((NP,), jnp.float32),
            pltpu.VMEM((CH, 128), jnp.int32),
            pltpu.VMEM((CH, 128), jnp.int32),
            pltpu.VMEM((3, 128), jnp.float32),
            pltpu.VMEM((3, 128, Fh), jnp.float32),
        ] + [pltpu.SemaphoreType.DMA] * 9,
    )
    def sc_layer(h2d, asrc_hbm, adst_hbm, srcI, dstI, parts, dens,
                 den_sh, out_sh, asrc_v, adst_v,
                 srcP, dstP, wbuf, rowbuf,
                 g0, g1, g2, s0, s1, s2, d0, d1, d2):
        gsems = (g0, g1, g2)
        ssems = (s0, s1, s2)
        dsems = (d0, d1, d2)
        cid = lax.axis_index("c")
        sid = lax.axis_index("s")
        wid = cid * NS + sid
        base = sid * nS
        coff = cid * off

        pltpu.sync_copy(asrc_hbm, asrc_v)
        pltpu.sync_copy(adst_hbm, adst_v)

        def zrow(i, _):
            for q in range(Fh // L):
                rowbuf[0, i, pl.ds(q * L, L)] = jnp.zeros((L,), jnp.float32)
            return 0
        lax.fori_loop(0, 128, zrow, 0)
        for q in range(128 // L):
            wbuf[0, pl.ds(q * L, L)] = jnp.zeros((L,), jnp.float32)
        for k in range(nS // 128):
            pltpu.sync_copy(rowbuf.at[0],
                            out_sh.at[pl.ds(base + k * 128, 128)])
            pltpu.sync_copy(wbuf.at[0],
                            den_sh.at[pl.ds(base + k * 128, 128)])
        plsc.subcore_barrier()

        def mx(i, m):
            return jnp.maximum(m, asrc_v[pl.ds(i * L, L)])
        m16 = lax.fori_loop(0, NP // L, mx,
                            jnp.full((L,), -3.0e38, jnp.float32))
        amax = m16[0]
        for i in range(1, L):
            amax = jnp.maximum(amax, m16[i])

        def weights(g, b):
            for q in range(128 // L):
                sv = srcP[g, pl.ds(q * L, L)] - coff
                dv = dstP[g, pl.ds(q * L, L)]
                a_s = plsc.load_gather(asrc_v, [sv])
                a_d = plsc.load_gather(adst_v, [dv])
                al = a_s + a_d
                al = jnp.maximum(al, NEG * al)
                sh = a_d + amax
                sh = jnp.maximum(sh, NEG * sh)
                wbuf[b, pl.ds(q * L, L)] = jnp.exp(al - sh)

        def scale_rows(b):
            def rgrp(t, _):
                wv = wbuf[b, pl.ds(t * L, L)]
                for rr in range(L):
                    row = t * L + rr
                    ws = wv[rr]
                    for q in range(Fh // L):
                        rowbuf[b, row, pl.ds(q * L, L)] = (
                            rowbuf[b, row, pl.ds(q * L, L)] * ws)
                return 0
            lax.fori_loop(0, 128 // L, rgrp, 0)

        def sweep_block(blk):
            pltpu.sync_copy(srcI.at[cid, blk], srcP)
            pltpu.sync_copy(dstI.at[blk], dstP)
            pltpu.async_copy(h2d.at[srcP.at[0]], rowbuf.at[0], gsems[0])

            def piped(g2, _):
                for b in range(3):
                    g = g2 * 3 + b
                    nb = (b + 1) % 3
                    @pl.when(g2 >= 1)
                    def _():
                        pltpu.make_async_copy(
                            wbuf.at[b], den_sh.at[dstP.at[g]],
                            dsems[b]).wait()
                    weights(g, b)
                    pltpu.async_copy(wbuf.at[b], den_sh.at[dstP.at[g]],
                                     dsems[b], add=True)

                    def wait_row_scatter():
                        pltpu.make_async_copy(
                            rowbuf.at[nb], out_sh.at[dstP.at[g]],
                            ssems[nb]).wait()

                    def issue_gather():
                        pltpu.async_copy(h2d.at[srcP.at[g + 1]],
                                         rowbuf.at[nb], gsems[nb])
                    if b < 2:
                        @pl.when(g2 >= 1)
                        def _():
                            wait_row_scatter()
                        issue_gather()
                    else:
                        @pl.when(g2 < CH // 3 - 1)
                        def _():
                            wait_row_scatter()
                            issue_gather()

                    pltpu.make_async_copy(h2d.at[srcP.at[g]],
                                          rowbuf.at[b], gsems[b]).wait()
                    scale_rows(b)
                    pltpu.async_copy(rowbuf.at[b], out_sh.at[dstP.at[g]],
                                     ssems[b], add=True)
                return 0
            lax.fori_loop(0, CH // 3, piped, 0)

            for b in range(3):
                pltpu.make_async_copy(rowbuf.at[b],
                                      out_sh.at[dstP.at[0]],
                                      ssems[b]).wait()
                pltpu.make_async_copy(wbuf.at[b],
                                      den_sh.at[dstP.at[0]],
                                      dsems[b]).wait()

        if bpt == 1:
            sweep_block(wid)
        else:
            for bb in range(bpt):
                sweep_block(bpt * sid + bb)
        plsc.subcore_barrier()

        pltpu.sync_copy(out_sh.at[pl.ds(base, nS)],
                        parts.at[cid, pl.ds(base, nS)])
        pltpu.sync_copy(den_sh.at[pl.ds(base, nS)],
                        dens.at[cid, pl.ds(base, nS)])

    return sc_layer


def kernel(x, edge_index, W1, att_src1, att_dst1, b1,
           W2, att_src2, att_dst2, b2):
    N, Fin = x.shape
    E = edge_index.shape[1]
    H = W1.shape[1]
    C = W2.shape[1]

    NP = (N // (NS * 128) + 1) * (NS * 128)
    Et = E + N
    EP = -(-Et // (NW * 384)) * (NW * 384)
    CH = EP // (NW * 128)
    npad = EP - Et

    ei = edge_index.astype(jnp.int32)
    loops = jnp.arange(N, dtype=jnp.int32)
    pad_src = jnp.arange(npad, dtype=jnp.int32) % N
    pad_dst = N + jnp.arange(npad, dtype=jnp.int32) % (NP - N)
    src = jnp.concatenate([ei[0], loops, pad_src]).reshape(NW, CH, 128)
    dst = jnp.concatenate([ei[1], loops, pad_dst]).reshape(NW, CH, 128)
    src1 = jnp.stack([src, src + NP])
    src2 = jnp.stack([src, src])

    xp = jnp.zeros((NP, Fin), jnp.float32).at[:N].set(x)
    A21 = jnp.stack([att_src1, att_dst1], axis=1)
    A22 = jnp.stack([att_src2, att_dst2], axis=1)

    h1s, ap1 = _tc_prologue(xp, W1, A21)
    sc1 = _make_sc_layer(NP, H // 2, CH, bpt=2, off=NP)
    parts1, dens1 = sc1(h1s.reshape(2 * NP, H // 2),
                        ap1[:, 0], ap1[:, 1], src1, dst)

    h2, ap2 = _tc_mid(parts1, dens1, b1.reshape(1, H), W2, A22, N)
    sc2 = _make_sc_layer(NP, C, CH, bpt=1, off=0)
    parts2, dens2 = sc2(h2, ap2[:, 0], ap2[:, 1], src2, dst)

    out = _tc_final(parts2, dens2, b2.reshape(1, C))
    return out[:N], edge_index
